# baseline (speedup 1.0000x reference)
import math

import jax
import jax.numpy as jnp
from jax.experimental import pallas as pl
from jax.experimental.pallas import tpu as pltpu

D_MODEL = 608
HEADS = 8
DK = D_MODEL // HEADS
D_FF = 2048
B = 8
S = 43
NQ = 13
K_TOP = 4
N_BIG = 9


def _dot1(a, b, dims):
    return jax.lax.dot_general(a.astype(jnp.bfloat16), b.astype(jnp.bfloat16),
                               dims, preferred_element_type=jnp.float32)


def _nt(a, b):
    return _dot1(a, b, (((1,), (1,)), ((), ())))


def _nn(a, b):
    return _dot1(a, b, (((1,), (0,)), ((), ())))


def _ln(x, a, b, eps=1e-6):
    m = jnp.mean(x, axis=1, keepdims=True)
    xc = x - m
    var = jnp.sum(xc * xc, axis=1, keepdims=True) / (D_MODEL - 1)
    std = jnp.sqrt(var)
    return a * xc / (std + eps) + b


def _softmax(s):
    m = jnp.max(s, axis=-1, keepdims=True)
    e = jnp.exp(s - m)
    return e / jnp.sum(e, axis=-1, keepdims=True)


def _enc_layer(x, n1a, n1b, bq, bk, bv, bo, n2a, n2b, b1, b2,
               tqkv, two, tw1, tw2):
    xn = _ln(x, n1a, n1b)
    wqkv = tqkv()
    q = _nt(xn, wqkv[0:D_MODEL]) + bq
    k = _nt(xn, wqkv[D_MODEL:2 * D_MODEL]) + bk
    v = _nt(xn, wqkv[2 * D_MODEL:3 * D_MODEL]) + bv
    scale = 1.0 / math.sqrt(DK)
    d = lambda a, b, dims: jax.lax.dot_general(
        a, b, dims, preferred_element_type=jnp.float32)
    outs = []
    for bi in range(B):
        r0 = bi * S
        qb = q[r0:r0 + S].astype(jnp.bfloat16)
        kb = k[r0:r0 + S].astype(jnp.bfloat16)
        vb = v[r0:r0 + S].astype(jnp.bfloat16)
        ohs = []
        for h in range(HEADS):
            c0 = h * DK
            qh = qb[:, c0:c0 + DK]
            kh = kb[:, c0:c0 + DK]
            vh = vb[:, c0:c0 + DK]
            sc = d(qh, kh, (((1,), (1,)), ((), ()))) * scale
            p = _softmax(sc)
            ohs.append(d(p.astype(jnp.bfloat16), vh,
                         (((1,), (0,)), ((), ()))))
        outs.append(jnp.concatenate(ohs, axis=1))
    attn = jnp.concatenate(outs, axis=0)
    x = x + _nt(attn, two()) + bo
    xn2 = _ln(x, n2a, n2b)
    hmid = jnp.maximum(_nt(xn2, tw1()) + b1, 0.0)
    x = x + _nt(hmid, tw2()) + b2
    return x


def _final(x, n1a, n1b, bq, bk, tqk, wa, wb, ffb):
    xn = _ln(x, n1a, n1b)
    wqk = tqk()
    q = _nt(xn, wqk[0:D_MODEL]) + bq
    k = _nt(xn, wqk[D_MODEL:2 * D_MODEL]) + bk
    scale = 1.0 / math.sqrt(D_MODEL)
    nl = NQ * 4
    ic = jax.lax.broadcasted_iota(jnp.int32, (NQ, NQ), 1)
    la4 = jax.lax.broadcasted_iota(jnp.int32, (4, nl), 1)
    sa4 = jax.lax.broadcasted_iota(jnp.int32, (4, nl), 0)
    ta = (la4 // NQ == sa4).astype(jnp.float32)
    la13 = jax.lax.broadcasted_iota(jnp.int32, (NQ, nl), 1)
    sa13 = jax.lax.broadcasted_iota(jnp.int32, (NQ, nl), 0)
    th = (la13 % NQ == sa13).astype(jnp.float32)
    lane52 = jax.lax.broadcasted_iota(jnp.int32, (1, nl), 1)
    base52 = (lane52 < NQ).astype(jnp.float32)
    outs = []
    for bi in range(B):
        r0 = bi * S
        sc = _nt(q[r0:r0 + NQ], k[r0:r0 + S]) * scale
        p = _softmax(sc)
        imp = p[:, :NQ]
        rank = jnp.zeros((NQ, NQ), jnp.float32)
        for cp in range(NQ):
            vcp = imp[:, cp:cp + 1]
            beats = (vcp > imp) | ((vcp == imp) & (ic > cp))
            rank = rank + beats.astype(jnp.float32)
        hitf = (rank < K_TOP).astype(jnp.float32)
        x13 = x[r0:r0 + NQ]
        a = _nt(x13, wa) + ffb
        gt = _nt(wb, x13)
        g52 = jnp.concatenate([gt[d:d + 1, :] for d in range(4)], axis=1)
        aterm = jnp.dot(a, ta, preferred_element_type=jnp.float32)
        hitrep = jnp.dot(hitf, th, preferred_element_type=jnp.float32)
        out52 = base52 + hitrep * (aterm + g52 - base52)
        outs.append(out52)
    return jnp.stack(outs, axis=0)


def _body(*refs):
    x_ref = refs[0]
    big_hbm = refs[1:1 + N_BIG]
    sm = [r[...] for r in refs[1 + N_BIG:1 + N_BIG + 27]]
    out_ref = refs[1 + N_BIG + 27]
    big_vmem = refs[1 + N_BIG + 28:1 + N_BIG + 28 + N_BIG]
    sem = refs[-1]

    copies = [pltpu.make_async_copy(big_hbm[i], big_vmem[i], sem.at[i])
              for i in range(N_BIG)]
    for c in copies:
        c.start()
    waited = [False] * N_BIG

    def wget(i):
        def thunk():
            if not waited[i]:
                copies[i].wait()
                waited[i] = True
            return big_vmem[i][...]
        return thunk

    x = x_ref[...]
    for li in range(2):
        s0 = li * 10
        w0 = li * 4
        x = _enc_layer(x, sm[s0], sm[s0 + 1], sm[s0 + 2], sm[s0 + 3],
                       sm[s0 + 4], sm[s0 + 5], sm[s0 + 6], sm[s0 + 7],
                       sm[s0 + 8], sm[s0 + 9],
                       wget(w0), wget(w0 + 1), wget(w0 + 2), wget(w0 + 3))
    out_ref[...] = _final(x, sm[20], sm[21], sm[22], sm[23],
                          wget(8), sm[24], sm[25], sm[26])


def _unflatten(out52):
    return out52.reshape(B, NQ, 4, NQ).transpose(0, 1, 3, 2)


def kernel(src, mask, params):
    del mask
    x0 = src.reshape(B * S, D_MODEL)
    bf = jnp.bfloat16
    big = []
    smalls = []
    for p in params['layers']:
        big += [jnp.concatenate([p['Wq'], p['Wk'], p['Wv']], 0).astype(bf),
                p['Wo'].astype(bf), p['W1'].astype(bf), p['W2'].astype(bf)]
        smalls += [p['n1_a'].reshape(1, -1), p['n1_b'].reshape(1, -1),
                   p['bq'].reshape(1, -1), p['bk'].reshape(1, -1),
                   p['bv'].reshape(1, -1), p['bo'].reshape(1, -1),
                   p['n2_a'].reshape(1, -1), p['n2_b'].reshape(1, -1),
                   p['b1'].reshape(1, -1), p['b2'].reshape(1, -1)]
    pl_ = params['last']
    big += [jnp.concatenate([pl_['Wq'], pl_['Wk']], 0).astype(bf)]
    smalls += [pl_['n1_a'].reshape(1, -1), pl_['n1_b'].reshape(1, -1),
               pl_['bq'].reshape(1, -1), pl_['bk'].reshape(1, -1)]
    smalls += [params['ff_w'][:, :D_MODEL], params['ff_w'][:, D_MODEL:],
               params['ff_b'].reshape(1, -1)]
    vspec = pl.BlockSpec(memory_space=pltpu.VMEM)
    aspec = pl.BlockSpec(memory_space=pl.ANY)
    out52 = pl.pallas_call(
        _body,
        in_specs=[vspec] + [aspec] * N_BIG + [vspec] * len(smalls),
        out_specs=vspec,
        out_shape=jax.ShapeDtypeStruct((B, NQ, NQ * 4), jnp.float32),
        scratch_shapes=[pltpu.VMEM(w.shape, jnp.bfloat16) for w in big]
        + [pltpu.SemaphoreType.DMA((N_BIG,))],
    )(x0, *big, *smalls)
    return _unflatten(out52)


# speedup vs baseline: 1.2282x; 1.2282x over previous
import math

import jax
import jax.numpy as jnp
from jax.experimental import pallas as pl
from jax.experimental.pallas import tpu as pltpu

D_MODEL = 608
HEADS = 8
DK = D_MODEL // HEADS
D_FF = 2048
B = 8
S = 43
NQ = 13
K_TOP = 4
N_BIG = 14


def _dot1(a, b, dims):
    return jax.lax.dot_general(a.astype(jnp.bfloat16), b.astype(jnp.bfloat16),
                               dims, preferred_element_type=jnp.float32)


def _nt(a, b):
    return _dot1(a, b, (((1,), (1,)), ((), ())))


def _nn(a, b):
    return _dot1(a, b, (((1,), (0,)), ((), ())))


def _ln(x, a, b, eps=1e-6):
    m = jnp.mean(x, axis=1, keepdims=True)
    xc = x - m
    var = jnp.sum(xc * xc, axis=1, keepdims=True) / (D_MODEL - 1)
    std = jnp.sqrt(var)
    return a * xc / (std + eps) + b


def _softmax(s):
    m = jnp.max(s, axis=-1, keepdims=True)
    e = jnp.exp(s - m)
    return e / jnp.sum(e, axis=-1, keepdims=True)


def _enc_layer(x, n1a, n1b, bq, bk, bv, bo, n2a, n2b, b1, b2,
               twq, twk, twv, two, tw1, tw2, head_masks):
    del head_masks
    xn = _ln(x, n1a, n1b)
    q = _nt(xn, twq()) + bq
    k = _nt(xn, twk()) + bk
    v = _nt(xn, twv()) + bv
    scale = 1.0 / math.sqrt(DK)
    d = lambda a, b, dims: jax.lax.dot_general(
        a, b, dims, preferred_element_type=jnp.float32)
    outs = []
    for bi in range(B):
        r0 = bi * S
        qb = q[r0:r0 + S].astype(jnp.bfloat16)
        kb = k[r0:r0 + S].astype(jnp.bfloat16)
        vb = v[r0:r0 + S].astype(jnp.bfloat16)
        ohs = []
        for h in range(HEADS):
            c0 = h * DK
            qh = qb[:, c0:c0 + DK]
            kh = kb[:, c0:c0 + DK]
            vh = vb[:, c0:c0 + DK]
            sc = d(qh, kh, (((1,), (1,)), ((), ()))) * scale
            p = _softmax(sc)
            ohs.append(d(p.astype(jnp.bfloat16), vh,
                         (((1,), (0,)), ((), ()))))
        outs.append(jnp.concatenate(ohs, axis=1))
    attn = jnp.concatenate(outs, axis=0)
    x = x + _nt(attn, two()) + bo
    xn2 = _ln(x, n2a, n2b)
    hmid = jnp.maximum(_nt(xn2, tw1()) + b1, 0.0)
    x = x + _nt(hmid, tw2()) + b2
    return x


def _final(x, n1a, n1b, bq, bk, twq, twk, wa, wb, ffb):
    xn = _ln(x, n1a, n1b)
    q = _nt(xn, twq()) + bq
    k = _nt(xn, twk()) + bk
    scale = 1.0 / math.sqrt(D_MODEL)
    nl = NQ * 4
    ic = jax.lax.broadcasted_iota(jnp.int32, (NQ, NQ), 1)
    la4 = jax.lax.broadcasted_iota(jnp.int32, (4, nl), 1)
    sa4 = jax.lax.broadcasted_iota(jnp.int32, (4, nl), 0)
    ta = (la4 // NQ == sa4).astype(jnp.float32)
    la13 = jax.lax.broadcasted_iota(jnp.int32, (NQ, nl), 1)
    sa13 = jax.lax.broadcasted_iota(jnp.int32, (NQ, nl), 0)
    th = (la13 % NQ == sa13).astype(jnp.float32)
    lane52 = jax.lax.broadcasted_iota(jnp.int32, (1, nl), 1)
    base52 = (lane52 < NQ).astype(jnp.float32)
    outs = []
    for bi in range(B):
        r0 = bi * S
        sc = _nt(q[r0:r0 + NQ], k[r0:r0 + S]) * scale
        p = _softmax(sc)
        imp = p[:, :NQ]
        rank = jnp.zeros((NQ, NQ), jnp.float32)
        for cp in range(NQ):
            vcp = imp[:, cp:cp + 1]
            beats = (vcp > imp) | ((vcp == imp) & (ic > cp))
            rank = rank + beats.astype(jnp.float32)
        hitf = (rank < K_TOP).astype(jnp.float32)
        x13 = x[r0:r0 + NQ]
        a = _nt(x13, wa) + ffb
        gt = _nt(wb, x13)
        g52 = jnp.concatenate([gt[d:d + 1, :] for d in range(4)], axis=1)
        aterm = jnp.dot(a, ta, preferred_element_type=jnp.float32)
        hitrep = jnp.dot(hitf, th, preferred_element_type=jnp.float32)
        out52 = base52 + hitrep * (aterm + g52 - base52)
        outs.append(out52)
    return jnp.stack(outs, axis=0)


def _body(*refs):
    x_ref = refs[0]
    big_hbm = refs[1:1 + N_BIG]
    sm = [r[...] for r in refs[1 + N_BIG:1 + N_BIG + 27]]
    out_ref = refs[1 + N_BIG + 27]
    big_vmem = refs[1 + N_BIG + 28:1 + N_BIG + 28 + N_BIG]
    sem = refs[-1]

    copies = [pltpu.make_async_copy(big_hbm[i], big_vmem[i], sem.at[i])
              for i in range(N_BIG)]
    for c in copies:
        c.start()
    waited = [False] * N_BIG

    def wget(i):
        def thunk():
            if not waited[i]:
                copies[i].wait()
                waited[i] = True
            return big_vmem[i][...]
        return thunk

    lane = jax.lax.broadcasted_iota(jnp.int32, (1, D_MODEL), 1)
    head_masks = [((lane >= h * DK) & (lane < (h + 1) * DK)).astype(jnp.float32)
                  for h in range(HEADS)]
    x = x_ref[...]
    for li in range(2):
        s0 = li * 10
        w0 = li * 6
        x = _enc_layer(x, sm[s0], sm[s0 + 1], sm[s0 + 2], sm[s0 + 3],
                       sm[s0 + 4], sm[s0 + 5], sm[s0 + 6], sm[s0 + 7],
                       sm[s0 + 8], sm[s0 + 9],
                       wget(w0), wget(w0 + 1), wget(w0 + 2), wget(w0 + 3),
                       wget(w0 + 4), wget(w0 + 5), head_masks)
    out_ref[...] = _final(x, sm[20], sm[21], sm[22], sm[23],
                          wget(12), wget(13), sm[24], sm[25], sm[26])


def _unflatten(out52):
    return out52.reshape(B, NQ, 4, NQ).transpose(0, 1, 3, 2)


def kernel(src, mask, params):
    del mask
    x0 = src.reshape(B * S, D_MODEL)
    big = []
    smalls = []
    for p in params['layers']:
        big += [p['Wq'], p['Wk'], p['Wv'], p['Wo'], p['W1'], p['W2']]
        smalls += [p['n1_a'].reshape(1, -1), p['n1_b'].reshape(1, -1),
                   p['bq'].reshape(1, -1), p['bk'].reshape(1, -1),
                   p['bv'].reshape(1, -1), p['bo'].reshape(1, -1),
                   p['n2_a'].reshape(1, -1), p['n2_b'].reshape(1, -1),
                   p['b1'].reshape(1, -1), p['b2'].reshape(1, -1)]
    pl_ = params['last']
    big += [pl_['Wq'], pl_['Wk']]
    smalls += [pl_['n1_a'].reshape(1, -1), pl_['n1_b'].reshape(1, -1),
               pl_['bq'].reshape(1, -1), pl_['bk'].reshape(1, -1)]
    smalls += [params['ff_w'][:, :D_MODEL], params['ff_w'][:, D_MODEL:],
               params['ff_b'].reshape(1, -1)]
    vspec = pl.BlockSpec(memory_space=pltpu.VMEM)
    aspec = pl.BlockSpec(memory_space=pl.ANY)
    out52 = pl.pallas_call(
        _body,
        in_specs=[vspec] + [aspec] * N_BIG + [vspec] * len(smalls),
        out_specs=vspec,
        out_shape=jax.ShapeDtypeStruct((B, NQ, NQ * 4), jnp.float32),
        scratch_shapes=[pltpu.VMEM(w.shape, jnp.float32) for w in big]
        + [pltpu.SemaphoreType.DMA((N_BIG,))],
    )(x0, *big, *smalls)
    return _unflatten(out52)


# speedup vs baseline: 1.8947x; 1.5428x over previous
import math

import jax
import jax.numpy as jnp
from jax.experimental import pallas as pl
from jax.experimental.pallas import tpu as pltpu

D_MODEL = 608
HEADS = 8
DK = D_MODEL // HEADS
D_FF = 2048
B = 8
S = 43
NQ = 13
K_TOP = 4
N_BIG = 14


def _dot1(a, b, dims):
    return jax.lax.dot_general(a.astype(jnp.bfloat16), b.astype(jnp.bfloat16),
                               dims, preferred_element_type=jnp.float32)


def _nt(a, b):
    return _dot1(a, b, (((1,), (1,)), ((), ())))


def _nn(a, b):
    return _dot1(a, b, (((1,), (0,)), ((), ())))


def _ln(x, a, b, eps=1e-6):
    m = jnp.mean(x, axis=1, keepdims=True)
    xc = x - m
    var = jnp.sum(xc * xc, axis=1, keepdims=True) / (D_MODEL - 1)
    std = jnp.sqrt(var)
    return a * xc / (std + eps) + b


def _softmax(s):
    m = jnp.max(s, axis=-1, keepdims=True)
    e = jnp.exp(s - m)
    return e / jnp.sum(e, axis=-1, keepdims=True)


def _enc_layer(x, n1a, n1b, bq, bk, bv, bo, n2a, n2b, b1, b2,
               twq, twk, twv, two, tw1, tw2, head_masks):
    del head_masks
    xn = _ln(x, n1a, n1b)
    q = _nt(xn, twq()) + bq
    k = _nt(xn, twk()) + bk
    v = _nt(xn, twv()) + bv
    scale = 1.0 / math.sqrt(DK)
    d = lambda a, b, dims: jax.lax.dot_general(
        a, b, dims, preferred_element_type=jnp.float32)
    outs = []
    for bi in range(B):
        r0 = bi * S
        qb = q[r0:r0 + S].astype(jnp.bfloat16)
        kb = k[r0:r0 + S].astype(jnp.bfloat16)
        vb = v[r0:r0 + S].astype(jnp.bfloat16)
        ohs = []
        for h in range(HEADS):
            c0 = h * DK
            qh = qb[:, c0:c0 + DK]
            kh = kb[:, c0:c0 + DK]
            vh = vb[:, c0:c0 + DK]
            sc = d(qh, kh, (((1,), (1,)), ((), ()))) * scale
            p = _softmax(sc)
            ohs.append(d(p.astype(jnp.bfloat16), vh,
                         (((1,), (0,)), ((), ()))))
        outs.append(jnp.concatenate(ohs, axis=1))
    attn = jnp.concatenate(outs, axis=0)
    x = x + _nt(attn, two()) + bo
    xn2 = _ln(x, n2a, n2b)
    hmid = jnp.maximum(_nt(xn2, tw1()) + b1, 0.0)
    x = x + _nt(hmid, tw2()) + b2
    return x


def _final(x, n1a, n1b, bq, bk, twq, twk, wa, wb, ffb):
    xn = _ln(x, n1a, n1b)
    q = _nt(xn, twq()) + bq
    k = _nt(xn, twk()) + bk
    scale = 1.0 / math.sqrt(D_MODEL)
    nr = NQ * NQ
    ic = jax.lax.broadcasted_iota(jnp.int32, (NQ, NQ), 1)
    s169 = jax.lax.broadcasted_iota(jnp.int32, (nr, NQ), 0)
    l13 = jax.lax.broadcasted_iota(jnp.int32, (nr, NQ), 1)
    rsel = (l13 == s169 // NQ).astype(jnp.float32)
    rgsel = (l13 == s169 % NQ).astype(jnp.float32)
    lane4 = jax.lax.broadcasted_iota(jnp.int32, (nr, 4), 1)
    base = (lane4 == 0).astype(jnp.float32)
    base_row = (jax.lax.broadcasted_iota(jnp.int32, (1, 4), 1) == 0
                ).astype(jnp.float32)
    fdot = lambda a, b: jnp.dot(a, b, preferred_element_type=jnp.float32)
    outs = []
    for bi in range(B):
        r0 = bi * S
        sc = _nt(q[r0:r0 + NQ], k[r0:r0 + S]) * scale
        p = _softmax(sc)
        imp = p[:, :NQ]
        rank = jnp.zeros((NQ, NQ), jnp.float32)
        for cp in range(NQ):
            vcp = imp[:, cp:cp + 1]
            beats = (vcp > imp) | ((vcp == imp) & (ic > cp))
            rank = rank + beats.astype(jnp.float32)
        hitf = (rank < K_TOP).astype(jnp.float32)
        x13 = x[r0:r0 + NQ]
        ap = _nt(x13, wa) + ffb - base_row
        g = _nt(x13, wb)
        valmb = fdot(rsel, ap) + fdot(rgsel, g)
        mid = fdot(rsel, hitf)
        hitcol = jnp.sum(mid * rgsel, axis=1, keepdims=True)
        outs.append(base + hitcol * valmb)
    return jnp.concatenate(outs, axis=0)


def _body(*refs):
    x_ref = refs[0]
    big_hbm = refs[1:1 + N_BIG]
    sm = [jnp.reshape(r[...], (1, -1)) for r in refs[1 + N_BIG:1 + N_BIG + 25]]
    ffw_ref = refs[1 + N_BIG + 25]
    out_ref = refs[1 + N_BIG + 26]
    big_vmem = refs[1 + N_BIG + 27:1 + N_BIG + 27 + N_BIG]
    sem = refs[-1]

    copies = [pltpu.make_async_copy(big_hbm[i], big_vmem[i], sem.at[i])
              for i in range(N_BIG)]
    for c in copies:
        c.start()
    waited = [False] * N_BIG

    def wget(i):
        def thunk():
            if not waited[i]:
                copies[i].wait()
                waited[i] = True
            return big_vmem[i][...]
        return thunk

    ffw = ffw_ref[...]
    wa = ffw[:, :D_MODEL]
    wb = ffw[:, D_MODEL:]
    x = x_ref[...]
    for li in range(2):
        s0 = li * 10
        w0 = li * 6
        x = _enc_layer(x, sm[s0], sm[s0 + 1], sm[s0 + 2], sm[s0 + 3],
                       sm[s0 + 4], sm[s0 + 5], sm[s0 + 6], sm[s0 + 7],
                       sm[s0 + 8], sm[s0 + 9],
                       wget(w0), wget(w0 + 1), wget(w0 + 2), wget(w0 + 3),
                       wget(w0 + 4), wget(w0 + 5), None)
    out_ref[...] = _final(x, sm[20], sm[21], sm[22], sm[23],
                          wget(12), wget(13), wa, wb, sm[24])


def kernel(src, mask, params):
    del mask
    x0 = src.reshape(B * S, D_MODEL)
    big = []
    smalls = []
    for p in params['layers']:
        big += [p['Wq'], p['Wk'], p['Wv'], p['Wo'], p['W1'], p['W2']]
        smalls += [p['n1_a'], p['n1_b'], p['bq'], p['bk'], p['bv'], p['bo'],
                   p['n2_a'], p['n2_b'], p['b1'], p['b2']]
    pl_ = params['last']
    big += [pl_['Wq'], pl_['Wk']]
    smalls += [pl_['n1_a'], pl_['n1_b'], pl_['bq'], pl_['bk']]
    smalls += [params['ff_b']]
    vspec = pl.BlockSpec(memory_space=pltpu.VMEM)
    aspec = pl.BlockSpec(memory_space=pl.ANY)
    out2 = pl.pallas_call(
        _body,
        in_specs=[vspec] + [aspec] * N_BIG + [vspec] * (len(smalls) + 1),
        out_specs=vspec,
        out_shape=jax.ShapeDtypeStruct((B * NQ * NQ, 4), jnp.float32),
        scratch_shapes=[pltpu.VMEM(w.shape, jnp.float32) for w in big]
        + [pltpu.SemaphoreType.DMA((N_BIG,))],
    )(x0, *big, *smalls, params['ff_w'])
    return out2.reshape(B, NQ, NQ, 4)


# speedup vs baseline: 2.0790x; 1.0973x over previous
import math

import jax
import jax.numpy as jnp
from jax.experimental import pallas as pl
from jax.experimental.pallas import tpu as pltpu

D_MODEL = 608
HEADS = 8
DK = D_MODEL // HEADS
D_FF = 2048
B = 8
S = 43
NQ = 13
K_TOP = 4
N_BIG = 14


def _dot1(a, b, dims):
    return jax.lax.dot_general(a.astype(jnp.bfloat16), b.astype(jnp.bfloat16),
                               dims, preferred_element_type=jnp.float32)


def _nt(a, b):
    return _dot1(a, b, (((1,), (1,)), ((), ())))


def _nn(a, b):
    return _dot1(a, b, (((1,), (0,)), ((), ())))


def _ln(x, a, b, eps=1e-6):
    m = jnp.mean(x, axis=1, keepdims=True)
    xc = x - m
    var = jnp.sum(xc * xc, axis=1, keepdims=True) / (D_MODEL - 1)
    std = jnp.sqrt(var)
    return a * xc / (std + eps) + b


def _softmax(s):
    m = jnp.max(s, axis=-1, keepdims=True)
    e = jnp.exp(s - m)
    return e / jnp.sum(e, axis=-1, keepdims=True)


def _enc_layer(x, n1a, n1b, bq, bk, bv, bo, n2a, n2b, b1, b2,
               twq, twk, twv, two, tw1, tw2, head_masks):
    del head_masks
    xn = _ln(x, n1a, n1b)
    q = _nt(xn, twq()) + bq
    k = _nt(xn, twk()) + bk
    v = _nt(xn, twv()) + bv
    scale = 1.0 / math.sqrt(DK)
    d = lambda a, b, dims: jax.lax.dot_general(
        a, b, dims, preferred_element_type=jnp.float32)
    vhs = []
    scs = []
    for bi in range(B):
        r0 = bi * S
        qb = q[r0:r0 + S].astype(jnp.bfloat16)
        kb = k[r0:r0 + S].astype(jnp.bfloat16)
        vb = v[r0:r0 + S].astype(jnp.bfloat16)
        for h in range(HEADS):
            c0 = h * DK
            scs.append(d(qb[:, c0:c0 + DK], kb[:, c0:c0 + DK],
                         (((1,), (1,)), ((), ()))) * scale)
            vhs.append(vb[:, c0:c0 + DK])
    ps = [_softmax(sc).astype(jnp.bfloat16) for sc in scs]
    outs = []
    for bi in range(B):
        ohs = [d(ps[bi * HEADS + h], vhs[bi * HEADS + h],
                 (((1,), (0,)), ((), ()))) for h in range(HEADS)]
        outs.append(jnp.concatenate(ohs, axis=1))
    attn = jnp.concatenate(outs, axis=0)
    x = x + _nt(attn, two()) + bo
    xn2 = _ln(x, n2a, n2b)
    hmid = jnp.maximum(_nt(xn2, tw1()) + b1, 0.0)
    x = x + _nt(hmid, tw2()) + b2
    return x


def _final(x, n1a, n1b, bq, bk, twq, twk, wa, wb, ffb):
    xn = _ln(x, n1a, n1b)
    xn13 = jnp.concatenate([xn[bi * S:bi * S + NQ] for bi in range(B)], 0)
    x13a = jnp.concatenate([x[bi * S:bi * S + NQ] for bi in range(B)], 0)
    q13 = _nt(xn13, twq()) + bq
    k = _nt(xn, twk()) + bk
    scale = 1.0 / math.sqrt(D_MODEL)
    nr = NQ * NQ
    ic = jax.lax.broadcasted_iota(jnp.int32, (NQ, NQ), 1)
    s169 = jax.lax.broadcasted_iota(jnp.int32, (nr, NQ), 0)
    l13 = jax.lax.broadcasted_iota(jnp.int32, (nr, NQ), 1)
    rsel = (l13 == s169 // NQ).astype(jnp.float32)
    rgsel = (l13 == s169 % NQ).astype(jnp.float32)
    lane4 = jax.lax.broadcasted_iota(jnp.int32, (nr, 4), 1)
    base = (lane4 == 0).astype(jnp.float32)
    base_row = (jax.lax.broadcasted_iota(jnp.int32, (1, 4), 1) == 0
                ).astype(jnp.float32)
    fdot = lambda a, b: jnp.dot(a, b, preferred_element_type=jnp.float32)
    imps = []
    for bi in range(B):
        sc = _nt(q13[bi * NQ:(bi + 1) * NQ], k[bi * S:bi * S + S]) * scale
        imps.append(_softmax(sc)[:, :NQ])
    imp3 = jnp.stack(imps, axis=0)
    ic3 = jax.lax.broadcasted_iota(jnp.int32, (B, NQ, NQ), 2)
    rank3 = jnp.zeros((B, NQ, NQ), jnp.float32)
    for cp in range(NQ):
        vcp = imp3[:, :, cp:cp + 1]
        beats = (vcp > imp3) | ((vcp == imp3) & (ic3 > cp))
        rank3 = rank3 + beats.astype(jnp.float32)
    hitf3 = (rank3 < K_TOP).astype(jnp.float32)
    ap_all = _nt(x13a, wa) + ffb - base_row
    g_all = _nt(x13a, wb)
    outs = []
    for bi in range(B):
        ap = ap_all[bi * NQ:(bi + 1) * NQ]
        g = g_all[bi * NQ:(bi + 1) * NQ]
        valmb = fdot(rsel, ap) + fdot(rgsel, g)
        mid = fdot(rsel, hitf3[bi])
        hitcol = jnp.sum(mid * rgsel, axis=1, keepdims=True)
        outs.append(base + hitcol * valmb)
    return jnp.concatenate(outs, axis=0)


def _body(*refs):
    x_ref = refs[0]
    big_hbm = refs[1:1 + N_BIG]
    sm = [jnp.reshape(r[...], (1, -1)) for r in refs[1 + N_BIG:1 + N_BIG + 25]]
    ffw_ref = refs[1 + N_BIG + 25]
    out_ref = refs[1 + N_BIG + 26]
    big_vmem = refs[1 + N_BIG + 27:1 + N_BIG + 27 + N_BIG]
    sem = refs[-1]

    copies = [pltpu.make_async_copy(big_hbm[i], big_vmem[i], sem.at[i])
              for i in range(N_BIG)]
    for c in copies:
        c.start()
    waited = [False] * N_BIG

    def wget(i):
        def thunk():
            if not waited[i]:
                copies[i].wait()
                waited[i] = True
            return big_vmem[i][...]
        return thunk

    ffw = ffw_ref[...]
    wa = ffw[:, :D_MODEL]
    wb = ffw[:, D_MODEL:]
    x = x_ref[...]
    for li in range(2):
        s0 = li * 10
        w0 = li * 6
        x = _enc_layer(x, sm[s0], sm[s0 + 1], sm[s0 + 2], sm[s0 + 3],
                       sm[s0 + 4], sm[s0 + 5], sm[s0 + 6], sm[s0 + 7],
                       sm[s0 + 8], sm[s0 + 9],
                       wget(w0), wget(w0 + 1), wget(w0 + 2), wget(w0 + 3),
                       wget(w0 + 4), wget(w0 + 5), None)
    out_ref[...] = _final(x, sm[20], sm[21], sm[22], sm[23],
                          wget(12), wget(13), wa, wb, sm[24])


def kernel(src, mask, params):
    del mask
    x0 = src.reshape(B * S, D_MODEL)
    big = []
    smalls = []
    for p in params['layers']:
        big += [p['Wq'], p['Wk'], p['Wv'], p['Wo'], p['W1'], p['W2']]
        smalls += [p['n1_a'], p['n1_b'], p['bq'], p['bk'], p['bv'], p['bo'],
                   p['n2_a'], p['n2_b'], p['b1'], p['b2']]
    pl_ = params['last']
    big += [pl_['Wq'], pl_['Wk']]
    smalls += [pl_['n1_a'], pl_['n1_b'], pl_['bq'], pl_['bk']]
    smalls += [params['ff_b']]
    vspec = pl.BlockSpec(memory_space=pltpu.VMEM)
    aspec = pl.BlockSpec(memory_space=pl.ANY)
    out2 = pl.pallas_call(
        _body,
        in_specs=[vspec] + [aspec] * N_BIG + [vspec] * (len(smalls) + 1),
        out_specs=vspec,
        out_shape=jax.ShapeDtypeStruct((B * NQ * NQ, 4), jnp.float32),
        scratch_shapes=[pltpu.VMEM(w.shape, jnp.float32) for w in big]
        + [pltpu.SemaphoreType.DMA((N_BIG,))],
    )(x0, *big, *smalls, params['ff_w'])
    return out2.reshape(B, NQ, NQ, 4)
